# baseline (speedup 1.0000x reference)
import functools

import jax
import jax.numpy as jnp
from jax import lax
from jax.experimental import pallas as pl
from jax.experimental.pallas import tpu as pltpu
from jax.experimental.pallas import tpu_sc as plsc

D = 32
L = 16
NW = 32
WCOLS = 768
NBUCK = 6
QS = 208
SELCAP = 784
MCAP = 64

_SHUF_DNUMS = lax.GatherDimensionNumbers(
    offset_dims=(), collapsed_slice_dims=(0,), start_index_map=(0,))


def _shuffle(v, idx):
  return lax.gather(v, idx[:, None], _SHUF_DNUMS, (1,),
                    mode=lax.GatherScatterMode.PROMISE_IN_BOUNDS)


def _popcnt(mask):
  return plsc.all_reduce_population_count(mask)[0]


def _gather_body(uid_hbm, iid_hbm, ue_hbm, ie_hbm, ucols_hbm, icols_hbm,
                 uchunk_v, ichunk_v, ut_v, ub_v, it_v, ib_v,
                 ut2_v, ub2_v, it2_v, ib2_v,
                 mt_v, mb_v, uslab_v, islab_v, rowbuf_v, dummy_v, sem, sem2,
                 semring, semu, semi, *, b, n, rw):
  del sem2
  nwin = rw // WCOLS
  phys_end = ((n + 127) // 128) * 128
  nc = 2
  wid = lax.axis_index("s") * nc + lax.axis_index("c")
  lo = wid * rw
  hi = lo + rw

  lanes = lax.iota(jnp.int32, L)

  def chunk_body(ch, carry):
    uoff, ioff = carry
    pltpu.sync_copy(uid_hbm.at[pl.ds(ch * 2048, 2048)], uchunk_v)
    pltpu.sync_copy(iid_hbm.at[pl.ds(ch * 2048, 2048)], ichunk_v)

    def sel_body(k, carry):
      uoff, ioff = carry
      bvec = lanes + ch * 2048 + k * L
      tu = uchunk_v[pl.ds(k * L, L)]
      ti = ichunk_v[pl.ds(k * L, L)]
      mu = (tu >= lo) & (tu < hi)
      mi = (ti >= lo) & (ti < hi)
      nu = _popcnt(mu)
      ni = _popcnt(mi)
      plsc.store_compressed(ut_v.at[pl.ds(uoff, L)], tu, mask=mu)
      plsc.store_compressed(ub_v.at[pl.ds(uoff, L)], bvec, mask=mu)
      plsc.store_compressed(it_v.at[pl.ds(ioff, L)], ti, mask=mi)
      plsc.store_compressed(ib_v.at[pl.ds(ioff, L)], bvec, mask=mi)
      return uoff + nu, ioff + ni

    return lax.fori_loop(0, 2048 // L, sel_body, (uoff, ioff), unroll=False)

  ucnt, icnt = lax.fori_loop(0, b // 2048, chunk_body,
                             (jnp.int32(0), jnp.int32(0)), unroll=False)

  neg = jnp.full((L,), -1, jnp.int32)
  zero = jnp.zeros((L,), jnp.int32)
  dump = lanes + b
  ut_v[pl.ds(ucnt, L)] = neg
  ub_v[pl.ds(ucnt, L)] = dump
  it_v[pl.ds(icnt, L)] = neg
  ib_v[pl.ds(icnt, L)] = dump

  ngrp_u = (ucnt + L - 1) // L
  ngrp_i = (icnt + L - 1) // L

  def bucketize(tsrc_v, bsrc_v, tdst_v, bdst_v, ngrp):
    qlens = []
    for q in range(NBUCK):
      def q_body(g, off):
        tg = tsrc_v[pl.ds(g * L, L)]
        bg = bsrc_v[pl.ds(g * L, L)]
        m = (((tg - lo) >> 8) // (6144 >> 8)) == q
        nn = _popcnt(m)
        plsc.store_compressed(tdst_v.at[pl.ds(q * QS + off, L)], tg, mask=m)
        plsc.store_compressed(bdst_v.at[pl.ds(q * QS + off, L)], bg, mask=m)
        return off + nn
      qlen = lax.fori_loop(0, ngrp, q_body, jnp.int32(0), unroll=False)
      tdst_v[pl.ds(q * QS + qlen, L)] = zero - 1
      bdst_v[pl.ds(q * QS + qlen, L)] = dump
      qlens.append((qlen + L - 1) // L)
    return qlens

  uq = bucketize(ut_v, ub_v, ut2_v, ub2_v, ngrp_u)
  iq = bucketize(it_v, ib_v, it2_v, ib2_v, ngrp_i)

  def pick(vals, q):
    r = vals[NBUCK - 1]
    for qq in range(NBUCK - 2, -1, -1):
      r = jnp.where(q == qq, vals[qq], r)
    return r

  def wstart(w):
    return pl.multiple_of(
        jnp.minimum(lo + w * WCOLS, phys_end - WCOLS), 128)

  def issue(w):
    p = w & 1
    ws = wstart(w)
    pltpu.async_copy(ue_hbm.at[:, pl.ds(ws, WCOLS)], uslab_v.at[p],
                     semu.at[p])
    pltpu.async_copy(ie_hbm.at[:, pl.ds(ws, WCOLS)], islab_v.at[p],
                     semi.at[p])

  issue(jnp.int32(0))
  issue(jnp.int32(1))

  def win_body(s, gcarry):
    p = s & 1
    ws = wstart(s)
    we = ws + WCOLS
    q = jnp.minimum(s >> 3, NBUCK - 1)
    qbase = q * QS

    def match(tlist_v, blist_v, ngrp):
      def m_body(g, moff):
        tg = tlist_v[pl.ds(qbase + g * L, L)]
        bg = blist_v[pl.ds(qbase + g * L, L)]
        m = (tg >= ws) & (tg < we)
        nn = _popcnt(m)
        plsc.store_compressed(mt_v.at[pl.ds(moff, L)], tg - ws, mask=m)
        plsc.store_compressed(mb_v.at[pl.ds(moff, L)], bg, mask=m)
        return moff + nn
      return lax.fori_loop(0, ngrp, m_body, jnp.int32(0), unroll=False)

    def extract(slab_v, cols_hbm, mcnt, gcarry):
      def e_body(g, gcarry):
        gi, gw = gcarry
        cg = mt_v[pl.ds(g * L, L)]
        bg = mb_v[pl.ds(g * L, L)]
        slot = gi & 3
        @pl.when(gi >= 4)
        def _():
          pltpu.make_async_copy(
              ucols_hbm.at[pl.ds(0, L)], dummy_v, semring.at[slot]).wait()
        rb = rowbuf_v.at[slot]
        for d in range(D):
          vals = plsc.load_gather(slab_v, [jnp.full((L,), d, jnp.int32), cg])
          plsc.store_scatter(rb, [lanes, jnp.full((L,), d, jnp.int32)], vals)
        pltpu.async_copy(rb, cols_hbm.at[bg], semring.at[slot])
        return gi + 1, gw

      ngrp = (mcnt + L - 1) // L
      return lax.fori_loop(0, ngrp, e_body, gcarry, unroll=False)

    pltpu.make_async_copy(ue_hbm.at[:, pl.ds(ws, WCOLS)], uslab_v.at[p],
                          semu.at[p]).wait()
    mcnt_u = match(ut2_v, ub2_v, pick(uq, q))
    mt_v[pl.ds(mcnt_u, L)] = zero
    mb_v[pl.ds(mcnt_u, L)] = dump
    gcarry = extract(uslab_v.at[p], ucols_hbm, mcnt_u, gcarry)

    pltpu.make_async_copy(ie_hbm.at[:, pl.ds(ws, WCOLS)], islab_v.at[p],
                          semi.at[p]).wait()
    mcnt_i = match(it2_v, ib2_v, pick(iq, q))
    mt_v[pl.ds(mcnt_i, L)] = zero
    mb_v[pl.ds(mcnt_i, L)] = dump
    gcarry = extract(islab_v.at[p], icols_hbm, mcnt_i, gcarry)

    @pl.when(s + 2 < nwin)
    def _():
      issue(s + 2)
    return gcarry

  gi, gw = lax.fori_loop(0, nwin, win_body, (jnp.int32(0), jnp.int32(0)),
                         unroll=False)

  for slot in range(4):
    @pl.when(gi > slot)
    def _():
      pltpu.make_async_copy(
          ucols_hbm.at[pl.ds(0, L)], dummy_v, semring.at[slot]).wait()


def _dot_body(uid_hbm, iid_hbm, ucols_hbm, icols_hbm, ub_hbm, ib_hbm,
              mean_hbm, out_hbm, uidx_v, iidx_v, urows_v, irows_v,
              ubv_v, ibv_v, out_v, mean_v, sem, *, bpw):
  nc = 2
  wid = lax.axis_index("s") * nc + lax.axis_index("c")
  base = wid * bpw

  pltpu.sync_copy(uid_hbm.at[pl.ds(base, bpw)], uidx_v)
  pltpu.sync_copy(iid_hbm.at[pl.ds(base, bpw)], iidx_v)
  pltpu.sync_copy(mean_hbm, mean_v.at[pl.ds(0, 1)])

  cub = pltpu.async_copy(ub_hbm.at[uidx_v], ubv_v, sem)
  cib = pltpu.async_copy(ib_hbm.at[iidx_v], ibv_v, sem)

  lanes = lax.iota(jnp.int32, L)
  nch = bpw // 128

  def ch_body(ch, _):
    c0 = ch * 128
    cu = pltpu.async_copy(ucols_hbm.at[pl.ds(base + c0, 128)], urows_v, sem)
    ci = pltpu.async_copy(icols_hbm.at[pl.ds(base + c0, 128)], irows_v, sem)
    cu.wait()
    ci.wait()
    for cc in range(8):
      acc = jnp.zeros((L,), jnp.float32)
      for j in range(L):
        r = cc * L + j
        u0 = urows_v[r, pl.ds(0, L)]
        u1 = urows_v[r, pl.ds(L, L)]
        i0 = irows_v[r, pl.ds(0, L)]
        i1 = irows_v[r, pl.ds(L, L)]
        s = u0 * i0 + u1 * i1
        for k in (8, 4, 2, 1):
          s = s + _shuffle(s, lanes ^ k)
        acc = jnp.where(lanes == j, s, acc)
      out_v[pl.ds(c0 + cc * L, L)] = acc
    return _

  lax.fori_loop(0, nch, ch_body, 0, unroll=False)

  cub.wait()
  cib.wait()

  m = mean_v[pl.ds(0, L)][0]

  def sig_body(c, _):
    sl = pl.ds(c * L, L)
    z = out_v[sl] + ubv_v[sl] + ibv_v[sl] + m
    out_v[sl] = 1.0 / (1.0 + jnp.exp(-z))
    return _

  lax.fori_loop(0, bpw // L, sig_body, 0, unroll=False)

  pltpu.sync_copy(out_v, out_hbm.at[pl.ds(base, bpw)])


@jax.jit
def kernel(x, user_emb, user_bias, item_emb, item_bias, mean):
  b = x.shape[0]
  n = user_emb.shape[0]
  bpw = b // NW
  rw = ((n + NW - 1) // NW + WCOLS - 1) // WCOLS * WCOLS
  uid = x[:, 0]
  iid = x[:, 1]
  ue_t = user_emb.T
  ie_t = item_emb.T
  ubf = user_bias.reshape(-1)
  ibf = item_bias.reshape(-1)
  mesh = plsc.VectorSubcoreMesh(core_axis_name="c", subcore_axis_name="s")
  cparams = pltpu.CompilerParams(use_tc_tiling_on_sc=True,
                                 needs_layout_passes=False)

  k1 = functools.partial(
      pl.kernel,
      mesh=mesh,
      compiler_params=cparams,
      out_type=(jax.ShapeDtypeStruct((b + L, 128), jnp.float32),
                jax.ShapeDtypeStruct((b + L, 128), jnp.float32)),
      scratch_types=[
          pltpu.VMEM((2048,), jnp.int32),
          pltpu.VMEM((2048,), jnp.int32),
          pltpu.VMEM((SELCAP,), jnp.int32),
          pltpu.VMEM((SELCAP,), jnp.int32),
          pltpu.VMEM((SELCAP,), jnp.int32),
          pltpu.VMEM((SELCAP,), jnp.int32),
          pltpu.VMEM((NBUCK * QS,), jnp.int32),
          pltpu.VMEM((NBUCK * QS,), jnp.int32),
          pltpu.VMEM((NBUCK * QS,), jnp.int32),
          pltpu.VMEM((NBUCK * QS,), jnp.int32),
          pltpu.VMEM((MCAP + L,), jnp.int32),
          pltpu.VMEM((MCAP + L,), jnp.int32),
          pltpu.VMEM((2, D, WCOLS), jnp.float32),
          pltpu.VMEM((2, D, WCOLS), jnp.float32),
          pltpu.VMEM((4, L, 128), jnp.float32),
          pltpu.VMEM((L, 128), jnp.float32),
          pltpu.SemaphoreType.DMA,
          pltpu.SemaphoreType.DMA,
          pltpu.SemaphoreType.DMA((4,)),
          pltpu.SemaphoreType.DMA((2,)),
          pltpu.SemaphoreType.DMA((2,)),
      ],
  )(functools.partial(_gather_body, b=b, n=n, rw=rw))
  ucols, icols = k1(uid, iid, ue_t, ie_t)

  k2 = functools.partial(
      pl.kernel,
      mesh=mesh,
      compiler_params=cparams,
      out_type=jax.ShapeDtypeStruct((b,), jnp.float32),
      scratch_types=[
          pltpu.VMEM((bpw,), jnp.int32),
          pltpu.VMEM((bpw,), jnp.int32),
          pltpu.VMEM((128, 128), jnp.float32),
          pltpu.VMEM((128, 128), jnp.float32),
          pltpu.VMEM((bpw,), jnp.float32),
          pltpu.VMEM((bpw,), jnp.float32),
          pltpu.VMEM((bpw,), jnp.float32),
          pltpu.VMEM((L,), jnp.float32),
          pltpu.SemaphoreType.DMA,
      ],
  )(functools.partial(_dot_body, bpw=bpw))
  return k2(uid, iid, ucols, icols, ubf, ibf, mean)


# speedup vs baseline: 1.2717x; 1.2717x over previous
import functools

import jax
import jax.numpy as jnp
from jax import lax
from jax.experimental import pallas as pl
from jax.experimental.pallas import tpu as pltpu
from jax.experimental.pallas import tpu_sc as plsc

D = 32
L = 16
NW = 32
WCOLS = 1024
SELCAP = 784
MCAP = 64

_SHUF_DNUMS = lax.GatherDimensionNumbers(
    offset_dims=(), collapsed_slice_dims=(0,), start_index_map=(0,))


def _shuffle(v, idx):
  return lax.gather(v, idx[:, None], _SHUF_DNUMS, (1,),
                    mode=lax.GatherScatterMode.PROMISE_IN_BOUNDS)


def _popcnt(mask):
  return plsc.all_reduce_population_count(mask)[0]


def _gather_body(uid_hbm, iid_hbm, ue_hbm, ie_hbm, ucols_hbm, icols_hbm,
                 uidall_v, iidall_v, ut_v, ub_v, it_v, ib_v,
                 mt_v, mb_v, uslab_v, islab_v, rowbuf_v, dummy_v, sem, sem2,
                 semring, *, b, n, rw):
  del sem2
  nwin = rw // WCOLS
  phys_end = ((n + 127) // 128) * 128
  nc = 2
  wid = lax.axis_index("s") * nc + lax.axis_index("c")
  lo = wid * rw
  hi = lo + rw

  pltpu.sync_copy(uid_hbm, uidall_v)
  pltpu.sync_copy(iid_hbm, iidall_v)

  lanes = lax.iota(jnp.int32, L)

  def sel_body(k, carry):
    uoff, ioff = carry
    bvec = lanes + k * L
    tu = uidall_v[pl.ds(k * L, L)]
    ti = iidall_v[pl.ds(k * L, L)]
    mu = (tu >= lo) & (tu < hi)
    mi = (ti >= lo) & (ti < hi)
    nu = _popcnt(mu)
    ni = _popcnt(mi)
    plsc.store_compressed(ut_v.at[pl.ds(uoff, L)], tu, mask=mu)
    plsc.store_compressed(ub_v.at[pl.ds(uoff, L)], bvec, mask=mu)
    plsc.store_compressed(it_v.at[pl.ds(ioff, L)], ti, mask=mi)
    plsc.store_compressed(ib_v.at[pl.ds(ioff, L)], bvec, mask=mi)
    return uoff + nu, ioff + ni

  ucnt, icnt = lax.fori_loop(0, b // L, sel_body, (jnp.int32(0), jnp.int32(0)),
                             unroll=False)

  neg = jnp.full((L,), -1, jnp.int32)
  zero = jnp.zeros((L,), jnp.int32)
  dump = lanes + b
  ut_v[pl.ds(ucnt, L)] = neg
  ub_v[pl.ds(ucnt, L)] = dump
  it_v[pl.ds(icnt, L)] = neg
  ib_v[pl.ds(icnt, L)] = dump

  ngrp_u = (ucnt + L - 1) // L
  ngrp_i = (icnt + L - 1) // L

  def win_body(s, gcarry):
    wraw = lo + s * WCOLS

    def active(gcarry):
      gi, gw = gcarry
      ws = pl.multiple_of(jnp.minimum(wraw, phys_end - WCOLS), 128)
      we = ws + WCOLS
      cu = pltpu.async_copy(ue_hbm.at[:, pl.ds(ws, WCOLS)], uslab_v, sem)
      ci = pltpu.async_copy(ie_hbm.at[:, pl.ds(ws, WCOLS)], islab_v, sem)

      def match(tlist_v, blist_v, ngrp):
        def m_body(g, moff):
          tg = tlist_v[pl.ds(g * L, L)]
          bg = blist_v[pl.ds(g * L, L)]
          m = (tg >= ws) & (tg < we)
          nn = _popcnt(m)
          plsc.store_compressed(mt_v.at[pl.ds(moff, L)], tg - ws, mask=m)
          plsc.store_compressed(mb_v.at[pl.ds(moff, L)], bg, mask=m)
          return moff + nn
        return lax.fori_loop(0, ngrp, m_body, jnp.int32(0), unroll=False)

      def extract(slab_v, cols_hbm, mcnt, gcarry):
        def e_body(g, gcarry):
          gi, gw = gcarry
          cg = mt_v[pl.ds(g * L, L)]
          bg = mb_v[pl.ds(g * L, L)]
          slot = gi & 3
          @pl.when(gi >= 4)
          def _():
            pltpu.make_async_copy(
                ucols_hbm.at[pl.ds(0, L)], dummy_v, semring.at[slot]).wait()
          rb = rowbuf_v.at[slot]
          for d in range(D):
            vals = plsc.load_gather(slab_v, [jnp.full((L,), d, jnp.int32), cg])
            plsc.store_scatter(rb, [lanes, jnp.full((L,), d, jnp.int32)], vals)
          pltpu.async_copy(rb, cols_hbm.at[bg], semring.at[slot])
          return gi + 1, gw

        ngrp = (mcnt + L - 1) // L
        return lax.fori_loop(0, ngrp, e_body, gcarry, unroll=False)

      cu.wait()
      mcnt_u = match(ut_v, ub_v, ngrp_u)
      mt_v[pl.ds(mcnt_u, L)] = zero
      mb_v[pl.ds(mcnt_u, L)] = dump
      gcarry = extract(uslab_v, ucols_hbm, mcnt_u, gcarry)

      ci.wait()
      mcnt_i = match(it_v, ib_v, ngrp_i)
      mt_v[pl.ds(mcnt_i, L)] = zero
      mb_v[pl.ds(mcnt_i, L)] = dump
      gcarry = extract(islab_v, icols_hbm, mcnt_i, gcarry)
      return gcarry

    return lax.cond(wraw < n, active, lambda c: c, gcarry)

  gi, gw = lax.fori_loop(0, nwin, win_body, (jnp.int32(0), jnp.int32(0)),
                         unroll=False)

  for slot in range(4):
    @pl.when(gi > slot)
    def _():
      pltpu.make_async_copy(
          ucols_hbm.at[pl.ds(0, L)], dummy_v, semring.at[slot]).wait()


def _dot_body(uid_hbm, iid_hbm, ucols_hbm, icols_hbm, ub_hbm, ib_hbm,
              mean_hbm, out_hbm, uidx_v, iidx_v, urows_v, irows_v,
              ubv_v, ibv_v, out_v, mean_v, sem, *, bpw):
  nc = 2
  wid = lax.axis_index("s") * nc + lax.axis_index("c")
  base = wid * bpw

  pltpu.sync_copy(uid_hbm.at[pl.ds(base, bpw)], uidx_v)
  pltpu.sync_copy(iid_hbm.at[pl.ds(base, bpw)], iidx_v)
  pltpu.sync_copy(mean_hbm, mean_v.at[pl.ds(0, 1)])

  cub = pltpu.async_copy(ub_hbm.at[uidx_v], ubv_v, sem)
  cib = pltpu.async_copy(ib_hbm.at[iidx_v], ibv_v, sem)

  lanes = lax.iota(jnp.int32, L)
  nch = bpw // 128

  def ch_body(ch, _):
    c0 = ch * 128
    cu = pltpu.async_copy(ucols_hbm.at[pl.ds(base + c0, 128)], urows_v, sem)
    ci = pltpu.async_copy(icols_hbm.at[pl.ds(base + c0, 128)], irows_v, sem)
    cu.wait()
    ci.wait()
    for cc in range(8):
      acc = jnp.zeros((L,), jnp.float32)
      for j in range(L):
        r = cc * L + j
        u0 = urows_v[r, pl.ds(0, L)]
        u1 = urows_v[r, pl.ds(L, L)]
        i0 = irows_v[r, pl.ds(0, L)]
        i1 = irows_v[r, pl.ds(L, L)]
        s = u0 * i0 + u1 * i1
        for k in (8, 4, 2, 1):
          s = s + _shuffle(s, lanes ^ k)
        acc = jnp.where(lanes == j, s, acc)
      out_v[pl.ds(c0 + cc * L, L)] = acc
    return _

  lax.fori_loop(0, nch, ch_body, 0, unroll=False)

  cub.wait()
  cib.wait()

  m = mean_v[pl.ds(0, L)][0]

  def sig_body(c, _):
    sl = pl.ds(c * L, L)
    z = out_v[sl] + ubv_v[sl] + ibv_v[sl] + m
    out_v[sl] = 1.0 / (1.0 + jnp.exp(-z))
    return _

  lax.fori_loop(0, bpw // L, sig_body, 0, unroll=False)

  pltpu.sync_copy(out_v, out_hbm.at[pl.ds(base, bpw)])


@jax.jit
def kernel(x, user_emb, user_bias, item_emb, item_bias, mean):
  b = x.shape[0]
  n = user_emb.shape[0]
  bpw = b // NW
  rw = ((n + NW - 1) // NW + WCOLS - 1) // WCOLS * WCOLS
  uid = x[:, 0]
  iid = x[:, 1]
  ue_t = user_emb.T
  ie_t = item_emb.T
  ubf = user_bias.reshape(-1)
  ibf = item_bias.reshape(-1)
  mesh = plsc.VectorSubcoreMesh(core_axis_name="c", subcore_axis_name="s")
  cparams = pltpu.CompilerParams(use_tc_tiling_on_sc=True,
                                 needs_layout_passes=False)

  k1 = functools.partial(
      pl.kernel,
      mesh=mesh,
      compiler_params=cparams,
      out_type=(jax.ShapeDtypeStruct((b + L, 128), jnp.float32),
                jax.ShapeDtypeStruct((b + L, 128), jnp.float32)),
      scratch_types=[
          pltpu.VMEM((b,), jnp.int32),
          pltpu.VMEM((b,), jnp.int32),
          pltpu.VMEM((SELCAP,), jnp.int32),
          pltpu.VMEM((SELCAP,), jnp.int32),
          pltpu.VMEM((SELCAP,), jnp.int32),
          pltpu.VMEM((SELCAP,), jnp.int32),
          pltpu.VMEM((MCAP + L,), jnp.int32),
          pltpu.VMEM((MCAP + L,), jnp.int32),
          pltpu.VMEM((D, WCOLS), jnp.float32),
          pltpu.VMEM((D, WCOLS), jnp.float32),
          pltpu.VMEM((4, L, 128), jnp.float32),
          pltpu.VMEM((L, 128), jnp.float32),
          pltpu.SemaphoreType.DMA,
          pltpu.SemaphoreType.DMA,
          pltpu.SemaphoreType.DMA((4,)),
      ],
  )(functools.partial(_gather_body, b=b, n=n, rw=rw))
  ucols, icols = k1(uid, iid, ue_t, ie_t)

  k2 = functools.partial(
      pl.kernel,
      mesh=mesh,
      compiler_params=cparams,
      out_type=jax.ShapeDtypeStruct((b,), jnp.float32),
      scratch_types=[
          pltpu.VMEM((bpw,), jnp.int32),
          pltpu.VMEM((bpw,), jnp.int32),
          pltpu.VMEM((128, 128), jnp.float32),
          pltpu.VMEM((128, 128), jnp.float32),
          pltpu.VMEM((bpw,), jnp.float32),
          pltpu.VMEM((bpw,), jnp.float32),
          pltpu.VMEM((bpw,), jnp.float32),
          pltpu.VMEM((L,), jnp.float32),
          pltpu.SemaphoreType.DMA,
      ],
  )(functools.partial(_dot_body, bpw=bpw))
  return k2(uid, iid, ucols, icols, ubf, ibf, mean)


# speedup vs baseline: 1.2791x; 1.0058x over previous
import functools

import jax
import jax.numpy as jnp
from jax import lax
from jax.experimental import pallas as pl
from jax.experimental.pallas import tpu as pltpu
from jax.experimental.pallas import tpu_sc as plsc

D = 32
L = 16
NW = 32
WCOLS = 1024
SELCAP = 784
MCAP = 64

_SHUF_DNUMS = lax.GatherDimensionNumbers(
    offset_dims=(), collapsed_slice_dims=(0,), start_index_map=(0,))


def _shuffle(v, idx):
  return lax.gather(v, idx[:, None], _SHUF_DNUMS, (1,),
                    mode=lax.GatherScatterMode.PROMISE_IN_BOUNDS)


def _popcnt(mask):
  return plsc.all_reduce_population_count(mask)[0]


def _gather_body(uid_hbm, iid_hbm, ue_hbm, ie_hbm, ucols_hbm, icols_hbm,
                 uidall_v, iidall_v, ut_v, ub_v, it_v, ib_v,
                 mt_v, mb_v, uslab_v, islab_v, rowbuf_v, dummy_v, sem, sem2,
                 semring, semu, semi, *, b, n, rw):
  del sem2
  nwin = rw // WCOLS
  phys_end = ((n + 127) // 128) * 128
  nc = 2
  wid = lax.axis_index("s") * nc + lax.axis_index("c")
  lo = wid * rw
  hi = lo + rw

  pltpu.sync_copy(uid_hbm, uidall_v)
  pltpu.sync_copy(iid_hbm, iidall_v)

  lanes = lax.iota(jnp.int32, L)

  def sel_body(k, carry):
    uoff, ioff = carry
    bvec = lanes + k * L
    tu = uidall_v[pl.ds(k * L, L)]
    ti = iidall_v[pl.ds(k * L, L)]
    mu = (tu >= lo) & (tu < hi)
    mi = (ti >= lo) & (ti < hi)
    nu = _popcnt(mu)
    ni = _popcnt(mi)
    plsc.store_compressed(ut_v.at[pl.ds(uoff, L)], tu, mask=mu)
    plsc.store_compressed(ub_v.at[pl.ds(uoff, L)], bvec, mask=mu)
    plsc.store_compressed(it_v.at[pl.ds(ioff, L)], ti, mask=mi)
    plsc.store_compressed(ib_v.at[pl.ds(ioff, L)], bvec, mask=mi)
    return uoff + nu, ioff + ni

  ucnt, icnt = lax.fori_loop(0, b // L, sel_body, (jnp.int32(0), jnp.int32(0)),
                             unroll=False)

  neg = jnp.full((L,), -1, jnp.int32)
  zero = jnp.zeros((L,), jnp.int32)
  dump = lanes + b
  ut_v[pl.ds(ucnt, L)] = neg
  ub_v[pl.ds(ucnt, L)] = dump
  it_v[pl.ds(icnt, L)] = neg
  ib_v[pl.ds(icnt, L)] = dump

  ngrp_u = (ucnt + L - 1) // L
  ngrp_i = (icnt + L - 1) // L

  def wstart(w):
    return pl.multiple_of(
        jnp.minimum(lo + w * WCOLS, phys_end - WCOLS), 128)

  nwin_t = jnp.minimum((hi - lo + WCOLS - 1) // WCOLS,
                       (n - lo + WCOLS - 1) // WCOLS)

  pltpu.async_copy(ue_hbm.at[:, pl.ds(wstart(jnp.int32(0)), WCOLS)],
                   uslab_v, semu)
  pltpu.async_copy(ie_hbm.at[:, pl.ds(wstart(jnp.int32(0)), WCOLS)],
                   islab_v, semi)

  def win_body(s, gcarry):
    ws = wstart(s)
    we = ws + WCOLS

    def match(tlist_v, blist_v, ngrp):
      def m_body(g, moff):
        tg = tlist_v[pl.ds(g * L, L)]
        bg = blist_v[pl.ds(g * L, L)]
        m = (tg >= ws) & (tg < we)
        nn = _popcnt(m)
        plsc.store_compressed(mt_v.at[pl.ds(moff, L)], tg - ws, mask=m)
        plsc.store_compressed(mb_v.at[pl.ds(moff, L)], bg, mask=m)
        return moff + nn
      return lax.fori_loop(0, ngrp, m_body, jnp.int32(0), unroll=False)

    def extract(slab_v, cols_hbm, mcnt, gcarry):
      def e_body(g, gcarry):
        gi, gw = gcarry
        cg = mt_v[pl.ds(g * L, L)]
        bg = mb_v[pl.ds(g * L, L)]
        slot = gi & 3
        @pl.when(gi >= 4)
        def _():
          pltpu.make_async_copy(
              ucols_hbm.at[pl.ds(0, L)], dummy_v, semring.at[slot]).wait()
        rb = rowbuf_v.at[slot]
        for d in range(D):
          vals = plsc.load_gather(slab_v, [jnp.full((L,), d, jnp.int32), cg])
          plsc.store_scatter(rb, [lanes, jnp.full((L,), d, jnp.int32)], vals)
        pltpu.async_copy(rb, cols_hbm.at[bg], semring.at[slot])
        return gi + 1, gw

      ngrp = (mcnt + L - 1) // L
      return lax.fori_loop(0, ngrp, e_body, gcarry, unroll=False)

    pltpu.make_async_copy(ue_hbm.at[:, pl.ds(ws, WCOLS)], uslab_v,
                          semu).wait()
    mcnt_u = match(ut_v, ub_v, ngrp_u)
    mt_v[pl.ds(mcnt_u, L)] = zero
    mb_v[pl.ds(mcnt_u, L)] = dump
    gcarry = extract(uslab_v, ucols_hbm, mcnt_u, gcarry)

    @pl.when(s + 1 < nwin_t)
    def _():
      pltpu.async_copy(ue_hbm.at[:, pl.ds(wstart(s + 1), WCOLS)],
                       uslab_v, semu)

    pltpu.make_async_copy(ie_hbm.at[:, pl.ds(ws, WCOLS)], islab_v,
                          semi).wait()
    mcnt_i = match(it_v, ib_v, ngrp_i)
    mt_v[pl.ds(mcnt_i, L)] = zero
    mb_v[pl.ds(mcnt_i, L)] = dump
    gcarry = extract(islab_v, icols_hbm, mcnt_i, gcarry)

    @pl.when(s + 1 < nwin_t)
    def _():
      pltpu.async_copy(ie_hbm.at[:, pl.ds(wstart(s + 1), WCOLS)],
                       islab_v, semi)
    return gcarry

  gi, gw = lax.fori_loop(0, nwin_t, win_body, (jnp.int32(0), jnp.int32(0)),
                         unroll=False)

  for slot in range(4):
    @pl.when(gi > slot)
    def _():
      pltpu.make_async_copy(
          ucols_hbm.at[pl.ds(0, L)], dummy_v, semring.at[slot]).wait()


def _dot_body(uid_hbm, iid_hbm, ucols_hbm, icols_hbm, ub_hbm, ib_hbm,
              mean_hbm, out_hbm, uidx_v, iidx_v, urows_v, irows_v,
              ubv_v, ibv_v, out_v, mean_v, sem, *, bpw):
  nc = 2
  wid = lax.axis_index("s") * nc + lax.axis_index("c")
  base = wid * bpw

  pltpu.sync_copy(uid_hbm.at[pl.ds(base, bpw)], uidx_v)
  pltpu.sync_copy(iid_hbm.at[pl.ds(base, bpw)], iidx_v)
  pltpu.sync_copy(mean_hbm, mean_v.at[pl.ds(0, 1)])

  cub = pltpu.async_copy(ub_hbm.at[uidx_v], ubv_v, sem)
  cib = pltpu.async_copy(ib_hbm.at[iidx_v], ibv_v, sem)

  lanes = lax.iota(jnp.int32, L)
  nch = bpw // 128

  def ch_body(ch, _):
    c0 = ch * 128
    cu = pltpu.async_copy(ucols_hbm.at[pl.ds(base + c0, 128)], urows_v, sem)
    ci = pltpu.async_copy(icols_hbm.at[pl.ds(base + c0, 128)], irows_v, sem)
    cu.wait()
    ci.wait()
    for cc in range(8):
      acc = jnp.zeros((L,), jnp.float32)
      for j in range(L):
        r = cc * L + j
        u0 = urows_v[r, pl.ds(0, L)]
        u1 = urows_v[r, pl.ds(L, L)]
        i0 = irows_v[r, pl.ds(0, L)]
        i1 = irows_v[r, pl.ds(L, L)]
        s = u0 * i0 + u1 * i1
        for k in (8, 4, 2, 1):
          s = s + _shuffle(s, lanes ^ k)
        acc = jnp.where(lanes == j, s, acc)
      out_v[pl.ds(c0 + cc * L, L)] = acc
    return _

  lax.fori_loop(0, nch, ch_body, 0, unroll=False)

  cub.wait()
  cib.wait()

  m = mean_v[pl.ds(0, L)][0]

  def sig_body(c, _):
    sl = pl.ds(c * L, L)
    z = out_v[sl] + ubv_v[sl] + ibv_v[sl] + m
    out_v[sl] = 1.0 / (1.0 + jnp.exp(-z))
    return _

  lax.fori_loop(0, bpw // L, sig_body, 0, unroll=False)

  pltpu.sync_copy(out_v, out_hbm.at[pl.ds(base, bpw)])


@jax.jit
def kernel(x, user_emb, user_bias, item_emb, item_bias, mean):
  b = x.shape[0]
  n = user_emb.shape[0]
  bpw = b // NW
  rw = ((n + NW - 1) // NW + WCOLS - 1) // WCOLS * WCOLS
  uid = x[:, 0]
  iid = x[:, 1]
  ue_t = user_emb.T
  ie_t = item_emb.T
  ubf = user_bias.reshape(-1)
  ibf = item_bias.reshape(-1)
  mesh = plsc.VectorSubcoreMesh(core_axis_name="c", subcore_axis_name="s")
  cparams = pltpu.CompilerParams(use_tc_tiling_on_sc=True,
                                 needs_layout_passes=False)

  k1 = functools.partial(
      pl.kernel,
      mesh=mesh,
      compiler_params=cparams,
      out_type=(jax.ShapeDtypeStruct((b + L, 128), jnp.float32),
                jax.ShapeDtypeStruct((b + L, 128), jnp.float32)),
      scratch_types=[
          pltpu.VMEM((b,), jnp.int32),
          pltpu.VMEM((b,), jnp.int32),
          pltpu.VMEM((SELCAP,), jnp.int32),
          pltpu.VMEM((SELCAP,), jnp.int32),
          pltpu.VMEM((SELCAP,), jnp.int32),
          pltpu.VMEM((SELCAP,), jnp.int32),
          pltpu.VMEM((MCAP + L,), jnp.int32),
          pltpu.VMEM((MCAP + L,), jnp.int32),
          pltpu.VMEM((D, WCOLS), jnp.float32),
          pltpu.VMEM((D, WCOLS), jnp.float32),
          pltpu.VMEM((4, L, 128), jnp.float32),
          pltpu.VMEM((L, 128), jnp.float32),
          pltpu.SemaphoreType.DMA,
          pltpu.SemaphoreType.DMA,
          pltpu.SemaphoreType.DMA((4,)),
          pltpu.SemaphoreType.DMA,
          pltpu.SemaphoreType.DMA,
      ],
  )(functools.partial(_gather_body, b=b, n=n, rw=rw))
  ucols, icols = k1(uid, iid, ue_t, ie_t)

  k2 = functools.partial(
      pl.kernel,
      mesh=mesh,
      compiler_params=cparams,
      out_type=jax.ShapeDtypeStruct((b,), jnp.float32),
      scratch_types=[
          pltpu.VMEM((bpw,), jnp.int32),
          pltpu.VMEM((bpw,), jnp.int32),
          pltpu.VMEM((128, 128), jnp.float32),
          pltpu.VMEM((128, 128), jnp.float32),
          pltpu.VMEM((bpw,), jnp.float32),
          pltpu.VMEM((bpw,), jnp.float32),
          pltpu.VMEM((bpw,), jnp.float32),
          pltpu.VMEM((L,), jnp.float32),
          pltpu.SemaphoreType.DMA,
      ],
  )(functools.partial(_dot_body, bpw=bpw))
  return k2(uid, iid, ucols, icols, ubf, ibf, mean)
